# baseline (speedup 1.0000x reference)
import functools

import jax
import jax.numpy as jnp
from jax import lax
from jax.experimental import pallas as pl
from jax.experimental.pallas import tpu as pltpu
from jax.experimental.pallas import tpu_sc as plsc

_VOCAB = 25002
_EMB = 100
_B = 4096
_L = 200

_COL_BLK = 5120
_N_BLK = 5
_VPAD = _COL_BLK * _N_BLK

_NC = 2
_NS = 16
_NW = _NC * _NS
_ROWS_PER_W = _B // _NW
_RG = _ROWS_PER_W // 16


def _matvec_body(tabt_ref, wt_ref, b_ref, v_ref):
    prod = tabt_ref[...] * wt_ref[...]
    s = jnp.sum(prod, axis=0)
    v_ref[...] = (s + b_ref[0]) * (1.0 / _L)


def _tc_matvec(tabT, Wt, b):
    return pl.pallas_call(
        _matvec_body,
        grid=(_N_BLK,),
        in_specs=[
            pl.BlockSpec((_EMB, _COL_BLK), lambda i: (0, i)),
            pl.BlockSpec((_EMB, 1), lambda i: (0, 0)),
            pl.BlockSpec(memory_space=pltpu.SMEM),
        ],
        out_specs=pl.BlockSpec((_COL_BLK,), lambda i: (i,)),
        out_shape=jax.ShapeDtypeStruct((_VPAD,), jnp.float32),
    )(tabT, Wt, b)


def _sc_body(textt_hbm, v_hbm, out_hbm, text_v, v_v, out_v, sem_v, sem_t):
    wid = lax.axis_index("s") * _NC + lax.axis_index("c")
    base = wid * _ROWS_PER_W
    cp_v = pltpu.async_copy(v_hbm, v_v, sem_v)
    cp_t = pltpu.async_copy(
        textt_hbm.at[:, pl.ds(base, _ROWS_PER_W)], text_v, sem_t
    )
    cp_v.wait()
    cp_t.wait()

    def step(l, accs):
        new = []
        for rg in range(_RG):
            tok = text_v[l, pl.ds(rg * 16, 16)]
            new.append(accs[rg] + plsc.load_gather(v_v, [tok]))
        return tuple(new)

    zero = jnp.zeros((16,), jnp.float32)
    accs = lax.fori_loop(0, _L, step, (zero,) * _RG)
    for rg in range(_RG):
        out_v[pl.ds(rg * 16, 16)] = accs[rg]
    pltpu.sync_copy(out_v, out_hbm.at[pl.ds(base, _ROWS_PER_W)])


_sc_pool = functools.partial(
    pl.kernel,
    out_type=jax.ShapeDtypeStruct((_B,), jnp.float32),
    mesh=plsc.VectorSubcoreMesh(core_axis_name="c", subcore_axis_name="s"),
    compiler_params=pltpu.CompilerParams(needs_layout_passes=False),
    scratch_types=[
        pltpu.VMEM((_L, _ROWS_PER_W), jnp.int32),
        pltpu.VMEM((_VPAD,), jnp.float32),
        pltpu.VMEM((_ROWS_PER_W,), jnp.float32),
        pltpu.SemaphoreType.DMA,
        pltpu.SemaphoreType.DMA,
    ],
)(_sc_body)


def kernel(text, text_lengths, table, W, b):
    v = _tc_matvec(table.T, W.T, b)
    out = _sc_pool(text.T, v)
    return out.reshape(_B, 1)


# speedup vs baseline: 1.0714x; 1.0714x over previous
import functools

import jax
import jax.numpy as jnp
from jax import lax
from jax.experimental import pallas as pl
from jax.experimental.pallas import tpu as pltpu
from jax.experimental.pallas import tpu_sc as plsc

_VOCAB = 25002
_EMB = 100
_B = 4096
_L = 200

_COL_BLK = 5120
_N_BLK = 5
_VPAD = _COL_BLK * _N_BLK

_NC = 2
_NS = 16
_NW = _NC * _NS
_ROWS_PER_W = _B // _NW
_RG = _ROWS_PER_W // 16


def _matvec_body(tabt_ref, w_ref, b_ref, v_ref):
    s = lax.dot_general(
        w_ref[...], tabt_ref[...],
        (((1,), (0,)), ((), ())),
        preferred_element_type=jnp.float32,
    )
    v_ref[...] = (s[0] + b_ref[0]) * (1.0 / _L)


def _tc_matvec(tabT, W, b):
    return pl.pallas_call(
        _matvec_body,
        grid=(_N_BLK,),
        in_specs=[
            pl.BlockSpec((_EMB, _COL_BLK), lambda i: (0, i)),
            pl.BlockSpec((1, _EMB), lambda i: (0, 0)),
            pl.BlockSpec(memory_space=pltpu.SMEM),
        ],
        out_specs=pl.BlockSpec((_COL_BLK,), lambda i: (i,)),
        out_shape=jax.ShapeDtypeStruct((_VPAD,), jnp.float32),
    )(tabT, W, b)


_N_CHUNK = 5
_L_CHUNK = _L // _N_CHUNK


def _sc_body(textt_hbm, v_hbm, out_hbm, text_v, v_v, out_v, sem_v, *sem_t):
    wid = lax.axis_index("s") * _NC + lax.axis_index("c")
    base = wid * _ROWS_PER_W
    cp_v = pltpu.async_copy(v_hbm, v_v, sem_v)
    cps = [
        pltpu.async_copy(
            textt_hbm.at[pl.ds(k * _L_CHUNK, _L_CHUNK), pl.ds(base, _ROWS_PER_W)],
            text_v.at[pl.ds(k * _L_CHUNK, _L_CHUNK), :],
            sem_t[k],
        )
        for k in range(_N_CHUNK)
    ]
    cp_v.wait()

    def step(l, accs):
        new = []
        for rg in range(_RG):
            tok = text_v[l, pl.ds(rg * 16, 16)]
            new.append(accs[rg] + plsc.load_gather(v_v, [tok]))
        return tuple(new)

    accs = (jnp.zeros((16,), jnp.float32),) * _RG
    for k in range(_N_CHUNK):
        cps[k].wait()
        accs = lax.fori_loop(k * _L_CHUNK, (k + 1) * _L_CHUNK, step, accs)
    for rg in range(_RG):
        out_v[pl.ds(rg * 16, 16)] = accs[rg]
    pltpu.sync_copy(out_v, out_hbm.at[pl.ds(base, _ROWS_PER_W)])


_sc_pool = functools.partial(
    pl.kernel,
    out_type=jax.ShapeDtypeStruct((_B,), jnp.float32),
    mesh=plsc.VectorSubcoreMesh(core_axis_name="c", subcore_axis_name="s"),
    compiler_params=pltpu.CompilerParams(needs_layout_passes=False),
    scratch_types=[
        pltpu.VMEM((_L, _ROWS_PER_W), jnp.int32),
        pltpu.VMEM((_VPAD,), jnp.float32),
        pltpu.VMEM((_ROWS_PER_W,), jnp.float32),
        pltpu.SemaphoreType.DMA,
        pltpu.SemaphoreType.DMA,
        pltpu.SemaphoreType.DMA,
        pltpu.SemaphoreType.DMA,
        pltpu.SemaphoreType.DMA,
        pltpu.SemaphoreType.DMA,
    ],
)(_sc_body)


def kernel(text, text_lengths, table, W, b):
    v = _tc_matvec(table.T, W, b)
    out = _sc_pool(text.T, v)
    return out.reshape(_B, 1)


# speedup vs baseline: 1.1046x; 1.0310x over previous
import functools

import jax
import jax.numpy as jnp
from jax import lax
from jax.experimental import pallas as pl
from jax.experimental.pallas import tpu as pltpu
from jax.experimental.pallas import tpu_sc as plsc

_VOCAB = 25002
_EMB = 100
_B = 4096
_L = 200

_COL_BLK = 5120
_N_BLK = 5
_VPAD = _COL_BLK * _N_BLK

_NC = 2
_NS = 16
_NW = _NC * _NS
_ROWS_PER_W = _B // _NW
_RG = _ROWS_PER_W // 16


def _matvec_body(tabt_ref, w_ref, b_ref, v_ref):
    s = lax.dot_general(
        w_ref[...], tabt_ref[...],
        (((1,), (0,)), ((), ())),
        preferred_element_type=jnp.float32,
    )
    v_ref[...] = (s[0] + b_ref[0]) * (1.0 / _L)


def _tc_matvec(tabT, W, b):
    return pl.pallas_call(
        _matvec_body,
        grid=(1,),
        in_specs=[
            pl.BlockSpec((_EMB, _VPAD), lambda i: (0, 0)),
            pl.BlockSpec((1, _EMB), lambda i: (0, 0)),
            pl.BlockSpec(memory_space=pltpu.SMEM),
        ],
        out_specs=pl.BlockSpec((_VPAD,), lambda i: (0,)),
        out_shape=jax.ShapeDtypeStruct((_VPAD,), jnp.float32),
    )(tabT, W, b)


_N_CHUNK = 5
_L_CHUNK = _L // _N_CHUNK


def _sc_body(textt_hbm, v_hbm, out_hbm, text_v, v_v, out_v, sem_v, *sem_t):
    wid = lax.axis_index("s") * _NC + lax.axis_index("c")
    base = wid * _ROWS_PER_W
    cp_v = pltpu.async_copy(v_hbm, v_v, sem_v)
    cps = [
        pltpu.async_copy(
            textt_hbm.at[pl.ds(k * _L_CHUNK, _L_CHUNK), pl.ds(base, _ROWS_PER_W)],
            text_v.at[pl.ds(k * _L_CHUNK, _L_CHUNK), :],
            sem_t[k],
        )
        for k in range(_N_CHUNK)
    ]
    cp_v.wait()

    def step(l, accs):
        new = []
        for rg in range(_RG):
            tok = text_v[l, pl.ds(rg * 16, 16)]
            new.append(accs[rg] + plsc.load_gather(v_v, [tok]))
        return tuple(new)

    accs = (jnp.zeros((16,), jnp.float32),) * _RG
    for k in range(_N_CHUNK):
        cps[k].wait()
        accs = lax.fori_loop(k * _L_CHUNK, (k + 1) * _L_CHUNK, step, accs)
    for rg in range(_RG):
        out_v[pl.ds(rg * 16, 16)] = accs[rg]
    pltpu.sync_copy(out_v, out_hbm.at[pl.ds(base, _ROWS_PER_W)])


_sc_pool = functools.partial(
    pl.kernel,
    out_type=jax.ShapeDtypeStruct((_B,), jnp.float32),
    mesh=plsc.VectorSubcoreMesh(core_axis_name="c", subcore_axis_name="s"),
    compiler_params=pltpu.CompilerParams(needs_layout_passes=False),
    scratch_types=[
        pltpu.VMEM((_L, _ROWS_PER_W), jnp.int32),
        pltpu.VMEM((_VPAD,), jnp.float32),
        pltpu.VMEM((_ROWS_PER_W,), jnp.float32),
        pltpu.SemaphoreType.DMA,
        pltpu.SemaphoreType.DMA,
        pltpu.SemaphoreType.DMA,
        pltpu.SemaphoreType.DMA,
        pltpu.SemaphoreType.DMA,
        pltpu.SemaphoreType.DMA,
    ],
)(_sc_body)


def kernel(text, text_lengths, table, W, b):
    v = _tc_matvec(table.T, W, b)
    out = _sc_pool(text.T, v)
    return out.reshape(_B, 1)
